# baseline (speedup 1.0000x reference)
import functools

import jax
import jax.numpy as jnp
from jax import lax
from jax.experimental import pallas as pl
from jax.experimental.pallas import tpu as pltpu
from jax.experimental.pallas import tpu_sc as plsc

N_ROWS = 1000
D_MODEL = 64
NUM_WORKERS = 32
D_SPLIT = 8
D_HALF = D_MODEL // D_SPLIT
CHUNK_B = 4096
H_OCT = 8


@functools.lru_cache(maxsize=None)
def _make_kernel(batch, hist):
    assert batch % (CHUNK_B * NUM_WORKERS // D_SPLIT) == 0
    assert hist % H_OCT == 0
    octets = hist // H_OCT
    mesh = plsc.VectorSubcoreMesh(core_axis_name="c", subcore_axis_name="s")

    @functools.partial(
        pl.kernel,
        out_type=jax.ShapeDtypeStruct((hist, D_MODEL, batch), jnp.float32),
        mesh=mesh,
        scratch_types=[
            pltpu.VMEM((D_HALF * N_ROWS,), jnp.float32),
            pltpu.VMEM((H_OCT, CHUNK_B), jnp.int32),
            pltpu.VMEM((D_HALF, CHUNK_B), jnp.float32),
            pltpu.VMEM((D_HALF, CHUNK_B), jnp.float32),
            pltpu.SemaphoreType.DMA,
            pltpu.SemaphoreType.DMA,
        ],
        compiler_params=pltpu.CompilerParams(use_tc_tiling_on_sc=True,
                                             needs_layout_passes=False),
    )
    def gather_kernel(idx_hbm, table_hbm, out_hbm, table_v, idx_v,
                      stage_a, stage_b, sem_a, sem_b):
        wid = lax.axis_index("s") * 2 + lax.axis_index("c")
        dh = wid % D_SPLIT
        b0 = pl.multiple_of((wid // D_SPLIT) * CHUNK_B, CHUNK_B)
        d_base = pl.multiple_of(dh * (D_HALF * N_ROWS), 8)
        pltpu.sync_copy(table_hbm.at[pl.ds(d_base, D_HALF * N_ROWS)], table_v)
        d_out = pl.multiple_of(dh * D_HALF, D_HALF)

        stages = (stage_a, stage_b)
        sems = (sem_a, sem_b)
        depth = len(stages)

        def octet_body(o, carry):
            h0 = pl.multiple_of(o * H_OCT, H_OCT)
            pltpu.sync_copy(
                idx_hbm.at[pl.ds(h0, H_OCT), pl.ds(b0, CHUNK_B)], idx_v)
            for hh in range(H_OCT):
                p = hh % depth
                stage, sem = stages[p], sems[p]

                def drain(stage=stage, sem=sem):
                    pltpu.make_async_copy(
                        out_hbm.at[0, pl.ds(0, D_HALF), pl.ds(0, CHUNK_B)],
                        stage, sem).wait()

                if hh >= depth:
                    drain()
                else:
                    pl.when(o > 0)(drain)

                def load_bv(bv):
                    iv = idx_v[hh, pl.ds(bv * 16, 16)]
                    return tuple(
                        plsc.load_gather(table_v, [iv + d * N_ROWS])
                        for d in range(D_HALF))

                def store_bv(bv, vals, stage=stage):
                    for d in range(D_HALF):
                        stage[d, pl.ds(bv * 16, 16)] = vals[d]

                def bv_body(bv, carry2):
                    cur = load_bv(bv)
                    store_bv(bv - 1, carry2)
                    return cur

                last = lax.fori_loop(1, CHUNK_B // 16, bv_body, load_bv(0))
                store_bv(CHUNK_B // 16 - 1, last)
                pltpu.async_copy(
                    stage,
                    out_hbm.at[h0 + hh, pl.ds(d_out, D_HALF),
                               pl.ds(b0, CHUNK_B)],
                    sem)
            return carry

        lax.fori_loop(0, octets, octet_body, 0)

        for p in range(depth):
            pltpu.make_async_copy(
                out_hbm.at[0, pl.ds(0, D_HALF), pl.ds(0, CHUNK_B)],
                stages[p], sems[p]).wait()

    return gather_kernel


def kernel(tf_indices, table):
    batch, hist = tf_indices.shape
    idx_t = tf_indices.T.astype(jnp.int32)
    table_t = table.T.reshape(D_MODEL * N_ROWS)
    out_t = _make_kernel(batch, hist)(idx_t, table_t)
    return out_t.transpose(2, 0, 1)


# speedup vs baseline: 1.6544x; 1.6544x over previous
import functools

import jax
import jax.numpy as jnp
from jax import lax
from jax.experimental import pallas as pl
from jax.experimental.pallas import tpu as pltpu
from jax.experimental.pallas import tpu_sc as plsc

N_ROWS = 1000
D_MODEL = 64
NUM_WORKERS = 32
CHUNK_B = 256
H_OCT = 8


@functools.lru_cache(maxsize=None)
def _make_kernel(batch, hist):
    chunks = batch // CHUNK_B
    chunks_per_w = chunks // NUM_WORKERS
    octets = hist // H_OCT
    assert batch % (CHUNK_B * NUM_WORKERS) == 0 and hist % H_OCT == 0
    mesh = plsc.VectorSubcoreMesh(core_axis_name="c", subcore_axis_name="s")

    @functools.partial(
        pl.kernel,
        out_type=jax.ShapeDtypeStruct((hist, D_MODEL, batch), jnp.float32),
        mesh=mesh,
        scratch_types=[
            pltpu.VMEM((N_ROWS * D_MODEL,), jnp.float32),
            pltpu.VMEM((H_OCT, CHUNK_B), jnp.int32),
            pltpu.VMEM((D_MODEL, CHUNK_B), jnp.float32),
            pltpu.VMEM((D_MODEL, CHUNK_B), jnp.float32),
            pltpu.VMEM((D_MODEL, CHUNK_B), jnp.float32),
            pltpu.SemaphoreType.DMA,
            pltpu.SemaphoreType.DMA,
            pltpu.SemaphoreType.DMA,
        ],
        compiler_params=pltpu.CompilerParams(use_tc_tiling_on_sc=True,
                                             needs_layout_passes=False),
    )
    def gather_kernel(idx_hbm, table_hbm, out_hbm, table_v, idx_v,
                      stage_a, stage_b, stage_c, sem_a, sem_b, sem_c):
        wid = lax.axis_index("s") * 2 + lax.axis_index("c")
        pltpu.sync_copy(table_hbm, table_v)

        stages = (stage_a, stage_b, stage_c)
        sems = (sem_a, sem_b, sem_c)
        depth = len(stages)

        for c in range(chunks_per_w):
            b0 = pl.multiple_of((wid * chunks_per_w + c) * CHUNK_B, CHUNK_B)

            def octet_body(o, carry, c=c):
                h0 = pl.multiple_of(o * H_OCT, H_OCT)
                pltpu.sync_copy(
                    idx_hbm.at[pl.ds(h0, H_OCT), pl.ds(b0, CHUNK_B)], idx_v)
                for hh in range(H_OCT):
                    p = hh % depth
                    stage, sem = stages[p], sems[p]

                    def drain(stage=stage, sem=sem):
                        pltpu.make_async_copy(
                            out_hbm.at[0, :, pl.ds(0, CHUNK_B)], stage,
                            sem).wait()

                    if hh >= depth or c > 0:
                        drain()
                    else:
                        pl.when(o > 0)(drain)

                    def bv_body(bv, carry2, stage=stage):
                        G = 8
                        iv = idx_v[hh, pl.ds(bv * 16, 16)]
                        prev = None
                        for dg in range(0, D_MODEL, G):
                            cur = [
                                (d, plsc.load_gather(table_v,
                                                     [iv + d * N_ROWS]))
                                for d in range(dg, dg + G)
                            ]
                            if prev is not None:
                                for d, vals in prev:
                                    stage[d, pl.ds(bv * 16, 16)] = vals
                            prev = cur
                        for d, vals in prev:
                            stage[d, pl.ds(bv * 16, 16)] = vals
                        return carry2

                    lax.fori_loop(0, CHUNK_B // 16, bv_body, 0)
                    pltpu.async_copy(
                        stage, out_hbm.at[h0 + hh, :, pl.ds(b0, CHUNK_B)],
                        sem)
                return carry

            lax.fori_loop(0, octets, octet_body, 0)

        for p in range(depth):
            pltpu.make_async_copy(
                out_hbm.at[0, :, pl.ds(0, CHUNK_B)], stages[p],
                sems[p]).wait()

    return gather_kernel


def kernel(tf_indices, table):
    batch, hist = tf_indices.shape
    idx_t = tf_indices.T.astype(jnp.int32)
    table_t = table.T.reshape(D_MODEL * N_ROWS)
    out_t = _make_kernel(batch, hist)(idx_t, table_t)
    return out_t.transpose(2, 0, 1)


# speedup vs baseline: 1.7448x; 1.0546x over previous
import functools

import jax
import jax.numpy as jnp
from jax import lax
from jax.experimental import pallas as pl
from jax.experimental.pallas import tpu as pltpu
from jax.experimental.pallas import tpu_sc as plsc

N_ROWS = 1000
D_MODEL = 64
NUM_WORKERS = 32
CHUNK_B = 256
H_OCT = 8
GATHER_GROUP = 8


@functools.lru_cache(maxsize=None)
def _make_kernel(batch, hist):
    chunks = batch // CHUNK_B
    chunks_per_w = chunks // NUM_WORKERS
    octets = hist // H_OCT
    assert batch % (CHUNK_B * NUM_WORKERS) == 0 and hist % H_OCT == 0
    assert octets % 2 == 1
    mesh = plsc.VectorSubcoreMesh(core_axis_name="c", subcore_axis_name="s")

    @functools.partial(
        pl.kernel,
        out_type=jax.ShapeDtypeStruct((hist, D_MODEL, batch), jnp.float32),
        mesh=mesh,
        scratch_types=[
            pltpu.VMEM((N_ROWS * D_MODEL,), jnp.float32),
            pltpu.VMEM((H_OCT, CHUNK_B), jnp.int32),
            pltpu.VMEM((H_OCT, CHUNK_B), jnp.int32),
            pltpu.VMEM((D_MODEL, CHUNK_B), jnp.float32),
            pltpu.VMEM((D_MODEL, CHUNK_B), jnp.float32),
            pltpu.VMEM((D_MODEL, CHUNK_B), jnp.float32),
            pltpu.SemaphoreType.DMA,
            pltpu.SemaphoreType.DMA,
            pltpu.SemaphoreType.DMA,
            pltpu.SemaphoreType.DMA,
            pltpu.SemaphoreType.DMA,
        ],
        compiler_params=pltpu.CompilerParams(use_tc_tiling_on_sc=True,
                                             needs_layout_passes=False),
    )
    def gather_kernel(idx_hbm, table_hbm, out_hbm, table_v, idx_a, idx_b,
                      stage_a, stage_b, stage_c,
                      sem_a, sem_b, sem_c, isem_a, isem_b):
        wid = lax.axis_index("s") * 2 + lax.axis_index("c")
        pltpu.sync_copy(table_hbm, table_v)

        stages = (stage_a, stage_b, stage_c)
        sems = (sem_a, sem_b, sem_c)
        depth = len(stages)

        for c in range(chunks_per_w):
            b0 = pl.multiple_of((wid * chunks_per_w + c) * CHUNK_B, CHUNK_B)

            def start_idx(o, buf, isem):
                h0 = pl.multiple_of(o * H_OCT, H_OCT)
                pltpu.async_copy(
                    idx_hbm.at[pl.ds(h0, H_OCT), pl.ds(b0, CHUNK_B)],
                    buf, isem)

            def wait_idx(buf, isem):
                pltpu.make_async_copy(
                    idx_hbm.at[pl.ds(0, H_OCT), pl.ds(b0, CHUNK_B)],
                    buf, isem).wait()

            def process_octet(o, idx_v, always_drain, c=c):
                h0 = pl.multiple_of(o * H_OCT, H_OCT)
                for hh in range(H_OCT):
                    p = hh % depth
                    stage, sem = stages[p], sems[p]

                    def drain(stage=stage, sem=sem):
                        pltpu.make_async_copy(
                            out_hbm.at[0, :, pl.ds(0, CHUNK_B)], stage,
                            sem).wait()

                    if hh >= depth or c > 0 or always_drain is True:
                        drain()
                    else:
                        pl.when(always_drain)(drain)

                    def bv_body(bv, carry2, stage=stage, idx_v=idx_v):
                        iv = idx_v[hh, pl.ds(bv * 16, 16)]
                        prev = None
                        for dg in range(0, D_MODEL, GATHER_GROUP):
                            cur = [
                                (d, plsc.load_gather(table_v,
                                                     [iv + d * N_ROWS]))
                                for d in range(dg, dg + GATHER_GROUP)
                            ]
                            if prev is not None:
                                for d, vals in prev:
                                    stage[d, pl.ds(bv * 16, 16)] = vals
                            prev = cur
                        for d, vals in prev:
                            stage[d, pl.ds(bv * 16, 16)] = vals
                        return carry2

                    lax.fori_loop(0, CHUNK_B // 16, bv_body, 0)
                    pltpu.async_copy(
                        stage, out_hbm.at[h0 + hh, :, pl.ds(b0, CHUNK_B)],
                        sem)

            start_idx(0, idx_a, isem_a)

            def pair_body(i, carry):
                o0 = pl.multiple_of(i * 2, 2)
                start_idx(o0 + 1, idx_b, isem_b)
                wait_idx(idx_a, isem_a)
                process_octet(o0, idx_a, i > 0)
                start_idx(o0 + 2, idx_a, isem_a)
                wait_idx(idx_b, isem_b)
                process_octet(o0 + 1, idx_b, True)
                return carry

            lax.fori_loop(0, octets // 2, pair_body, 0)
            wait_idx(idx_a, isem_a)
            process_octet(octets - 1, idx_a, True)

        for p in range(depth):
            pltpu.make_async_copy(
                out_hbm.at[0, :, pl.ds(0, CHUNK_B)], stages[p],
                sems[p]).wait()

    return gather_kernel


def kernel(tf_indices, table):
    batch, hist = tf_indices.shape
    idx_t = tf_indices.T.astype(jnp.int32)
    table_t = table.T.reshape(D_MODEL * N_ROWS)
    out_t = _make_kernel(batch, hist)(idx_t, table_t)
    return out_t.transpose(2, 0, 1)
